# baseline (speedup 1.0000x reference)
import functools

import jax
import jax.numpy as jnp
from jax import lax
from jax.experimental import pallas as pl
from jax.experimental.pallas import tpu as pltpu
from jax.experimental.pallas import tpu_sc as plsc

DEC = 96
W = 192
T = 4096
B = 32
NUM = T - 2 * W
PADNUM = 3840
G = 3
NGROUP = PADNUM // (G * 128)
TILEW = (G + 1) * 128

_NC = 2
_NS = 16


def _tc_body(x_ref, dists_ref, kb_ref):
    for j in range(W):
        col = x_ref[:, T - W + j : T - W + j + 1]
        kb_ref[:, j * 128 : (j + 1) * 128] = jnp.broadcast_to(col, (B, 128))

    for g in range(NGROUP):
        base = g * G * 128
        accs = [jnp.zeros((B, 128), jnp.float32) for _ in range(G)]
        for jh, njl in ((0, 128), (1, 64)):
            tile = x_ref[:, pl.ds(base + 128 * jh, TILEW)]

            def body(jl, accs, tile=tile, jh=jh):
                sl = (TILEW - jl) % TILEW
                rolled = pltpu.roll(tile, sl, axis=1)
                kjb = kb_ref[:, pl.ds((128 * jh + jl) * 128, 128)]
                out = []
                for s in range(G):
                    d = rolled[:, s * 128 : (s + 1) * 128] - kjb
                    out.append(accs[s] + d * d)
                return out

            accs = jax.lax.fori_loop(0, njl, body, accs, unroll=8)
        for s in range(G):
            dists_ref[:, base + s * 128 : base + (s + 1) * 128] = accs[s] / W


_sc_mesh = plsc.VectorSubcoreMesh(core_axis_name="c", subcore_axis_name="s")


@functools.partial(
    pl.kernel,
    out_type=jax.ShapeDtypeStruct((B * DEC,), jnp.float32),
    mesh=_sc_mesh,
    scratch_types=[
        pltpu.VMEM((PADNUM,), jnp.float32),
        pltpu.VMEM((T,), jnp.float32),
        pltpu.VMEM((DEC,), jnp.float32),
        pltpu.VMEM((16,), jnp.float32),
        pltpu.VMEM((16,), jnp.int32),
    ],
    compiler_params=pltpu.CompilerParams(needs_layout_passes=False),
)
def _sc_argmin_gather(dists_hbm, x_hbm, out_hbm, d_v, x_v, o_v, tf_v, ti_v):
    b = lax.axis_index("s") * _NC + lax.axis_index("c")
    pltpu.sync_copy(dists_hbm.at[pl.ds(pl.multiple_of(b * PADNUM, 8), PADNUM)], d_v)
    pltpu.sync_copy(x_hbm.at[pl.ds(pl.multiple_of(b * T, 8), T)], x_v)
    lanes = lax.iota(jnp.int32, 16)

    def body(i, carry):
        mv, mi = carry
        v = d_v[pl.ds(i * 16, 16)]
        idx = lanes + i * 16
        p = v < mv
        return jnp.where(p, v, mv), jnp.where(p, idx, mi)

    mv, mi = lax.fori_loop(
        0, NUM // 16, body,
        (jnp.full((16,), jnp.inf, jnp.float32), jnp.zeros((16,), jnp.int32)),
    )
    gmin = mv
    for sft in (8, 4, 2, 1):
        tf_v[pl.ds(0, 16)] = gmin
        gmin = jnp.minimum(gmin, plsc.load_gather(tf_v, [(lanes + sft) & 15]))
    idxv = jnp.where(mv == gmin, mi, jnp.full((16,), NUM, jnp.int32))
    for sft in (8, 4, 2, 1):
        ti_v[pl.ds(0, 16)] = idxv
        idxv = jnp.minimum(idxv, plsc.load_gather(ti_v, [(lanes + sft) & 15]))
    start = idxv + W
    for cdx in range(DEC // 16):
        pos = start + lanes + cdx * 16
        o_v[pl.ds(cdx * 16, 16)] = plsc.load_gather(x_v, [pos])
    pltpu.sync_copy(o_v, out_hbm.at[pl.ds(pl.multiple_of(b * DEC, 8), DEC)])


def kernel(feats_in, X_in, feats_out):
    x = X_in[:, :, 0]
    dists = pl.pallas_call(
        _tc_body,
        in_specs=[pl.BlockSpec((B, T), lambda: (0, 0))],
        out_specs=pl.BlockSpec((B, PADNUM), lambda: (0, 0)),
        out_shape=jax.ShapeDtypeStruct((B, PADNUM), jnp.float32),
        scratch_shapes=[pltpu.VMEM((B, W * 128), jnp.float32)],
    )(x)
    out = _sc_argmin_gather(dists.reshape(-1), x.reshape(-1))
    return out.reshape(B, DEC, 1)
